# baseline (speedup 1.0000x reference)
import functools

import jax
import jax.numpy as jnp
from jax import lax
from jax.experimental import pallas as pl
from jax.experimental.pallas import tpu as pltpu
from jax.experimental.pallas import tpu_sc as plsc

_B, _L, _C = 1, 2048, 1024
_H, _POS = 8, 16
_MAXF, _MINF = 16.0, 1.0
_S = 33
_SP = 48
_IP = 40
_D = _C // _H
_BLK = 256
_NBLK = _L // _BLK
_NC, _NS = 2, 16
_NW = _NC * _NS
_RPW = _L // _NW
_HIGH = jax.lax.Precision.HIGHEST


def _attn_body(xb_ref, wall_ref, ball_ref, wkh_ref, attn_ref, idx_ref, ent_ref):
    i = pl.program_id(0)
    xb = xb_ref[...]
    p = (jnp.dot(xb, wall_ref[...], preferred_element_type=jnp.float32)
         + ball_ref[...])
    p = p * jax.nn.sigmoid(p)

    freq = jax.nn.sigmoid(p[:, 0:8]) * (_MAXF - _MINF) + _MINF
    phase = jnp.tanh(p[:, 8:16]) * _MAXF
    decay = jax.nn.sigmoid(p[:, 16:24]) * 9.5 + 0.5
    freq_avg = jnp.mean(freq, axis=1, keepdims=True)
    phase_avg = jnp.mean(phase, axis=1, keepdims=True)

    q = p[:, 128:256]
    qk = jnp.dot(q.astype(jnp.bfloat16), wkh_ref[...],
                 preferred_element_type=jnp.float32)

    centers = (jnp.float32(i * _BLK)
               + lax.broadcasted_iota(jnp.int32, (_BLK, 1), 0)
               .astype(jnp.float32))
    lane = lax.broadcasted_iota(jnp.int32, (1, _SP), 1)
    grid = lane.astype(jnp.float32) - 16.0
    sp = centers + grid * freq_avg + phase_avg
    valid = (sp >= 0.0) & (sp < float(_L)) & (lane < _S)
    idx_ref[...] = jnp.clip(sp.astype(jnp.int32), 0, _L - 1)[:, 0:_IP]

    absgrid = jnp.abs(grid)
    validf = valid.astype(jnp.float32)
    ent = jnp.zeros((1, 1), jnp.float32)
    for h in range(_H):
        rel = absgrid * freq[:, h:h + 1]
        logit = qk[:, h:h + 1] * rel
        lm = jnp.where(valid, logit, -jnp.inf)
        m = jnp.max(lm, axis=1, keepdims=True)
        e = jnp.exp(lm - m)
        a0 = e / jnp.sum(e, axis=1, keepdims=True)
        env = jnp.exp(-rel / jnp.maximum(decay[:, h:h + 1], 0.1))
        a1 = a0 * env * validf
        a = a1 / (jnp.sum(a1, axis=1, keepdims=True) + 1e-8)
        attn_ref[h] = a
        ent = ent + jnp.sum(a * jnp.log(a + 1e-8)).reshape(1, 1)

    prev = ent_ref[...]
    acc = jnp.where(i == 0, ent, prev + ent)
    ent_ref[...] = jnp.where(i == _NBLK - 1, acc * (1.0 / (_L * _H)), acc)


def _stage_a(x2, wall, ball, wkh):
    return pl.pallas_call(
        _attn_body,
        grid=(_NBLK,),
        in_specs=[
            pl.BlockSpec((_BLK, _C), lambda i: (i, 0)),
            pl.BlockSpec((_C, 256), lambda i: (0, 0)),
            pl.BlockSpec((1, 256), lambda i: (0, 0)),
            pl.BlockSpec((_D, _H), lambda i: (0, 0)),
        ],
        out_specs=[
            pl.BlockSpec((_H, _BLK, _SP), lambda ii: (0, ii, 0)),
            pl.BlockSpec((_BLK, _IP), lambda i: (i, 0)),
            pl.BlockSpec((1, 1), lambda i: (0, 0)),
        ],
        out_shape=[
            jax.ShapeDtypeStruct((_H, _L, _SP), jnp.float32),
            jax.ShapeDtypeStruct((_L, _IP), jnp.int32),
            jax.ShapeDtypeStruct((1, 1), jnp.float32),
        ],
    )(x2, wall, ball, wkh)


def _sc_combine_body(x_hbm, idx_hbm, attn_hbm, y_hbm,
                     idx_v, w_v, rows_v, out_v, sem0, sem1):
    wid = lax.axis_index("s") * _NC + lax.axis_index("c")
    base = wid * _RPW
    pltpu.sync_copy(idx_hbm.at[pl.ds(base * _IP, _RPW * _IP)], idx_v)
    for h in range(_H):
        pltpu.sync_copy(
            attn_hbm.at[pl.ds((h * _L + base) * _SP, _RPW * _SP)],
            w_v.at[pl.ds(h * _RPW * _SP, _RPW * _SP)])

    def idx_row(i):
        return idx_v.at[pl.ds(i * _IP, _IP)]

    def compute_row(i, slot):
        r = jnp.bitwise_and(i, 7)
        for h in range(_H):
            wbase = (h * _RPW) * _SP + i * _SP
            wv = [w_v[pl.ds(wbase + k * 16, 16)] for k in range(_SP // 16)]
            wsc = [wv[s // 16][s % 16] for s in range(_S)]

            def chunk(v, carry):
                offw = h * (_D // 2) + v * 16
                offo = h * _D + v * 32
                acca = [None] * 4
                accb = [None] * 4
                for s in range(_S):
                    pk = rows_v[slot, s, pl.ds(offw, 16)]
                    a = lax.bitcast_convert_type(pk << 16, jnp.float32)
                    b = lax.bitcast_convert_type(pk & jnp.int32(-65536), jnp.float32)
                    if s < 4:
                        acca[s] = wsc[s] * a
                        accb[s] = wsc[s] * b
                    else:
                        acca[s % 4] = acca[s % 4] + wsc[s] * a
                        accb[s % 4] = accb[s % 4] + wsc[s] * b
                out_v[r, pl.ds(offo, 16)] = ((acca[0] + acca[1])
                                             + (acca[2] + acca[3]))
                out_v[r, pl.ds(offo + 16, 16)] = ((accb[0] + accb[1])
                                                  + (accb[2] + accb[3]))
                return carry

            lax.fori_loop(0, _D // 32, chunk, 0, unroll=2)

    pltpu.async_copy(x_hbm.at[idx_row(0)], rows_v.at[0], sem0)

    def body2(j, carry):
        r0 = 2 * j
        pltpu.async_copy(x_hbm.at[idx_row(r0 + 1)], rows_v.at[1], sem1)
        pltpu.make_async_copy(x_hbm.at[idx_row(r0)], rows_v.at[0],
                              sem0).wait()
        compute_row(r0, 0)

        @pl.when(j < _RPW // 2 - 1)
        def _():
            pltpu.async_copy(x_hbm.at[idx_row(r0 + 2)], rows_v.at[0], sem0)

        pltpu.make_async_copy(x_hbm.at[idx_row(r0 + 1)], rows_v.at[1],
                              sem1).wait()
        compute_row(r0 + 1, 1)

        @pl.when(jnp.bitwise_and(j, 3) == 3)
        def _():
            pltpu.sync_copy(
                out_v,
                y_hbm.at[pl.ds(pl.multiple_of(base + r0 - 6, 8), 8)])

        return carry

    lax.fori_loop(0, _RPW // 2, body2, 0)


def _stage_b(x_pk, idx, attn8):
    mesh = plsc.VectorSubcoreMesh(core_axis_name="c", subcore_axis_name="s")
    fn = functools.partial(
        pl.kernel,
        mesh=mesh,
        out_type=jax.ShapeDtypeStruct((_L, _C), jnp.float32),
        scratch_types=[
            pltpu.VMEM((_RPW * _IP,), jnp.int32),
            pltpu.VMEM((_H * _RPW * _SP,), jnp.float32),
            pltpu.VMEM((2, _IP, _C // 2), jnp.int32),
            pltpu.VMEM((8, _C), jnp.float32),
            pltpu.SemaphoreType.DMA,
            pltpu.SemaphoreType.DMA,
        ],
    )(_sc_combine_body)
    return fn(x_pk, idx.reshape(_L * _IP), attn8.reshape(_H * _L * _SP))


def _mlp_body(yb_ref, w1t_ref, b1_ref, w2t_ref, b2_ref, woutt_ref, ob_ref):
    y = yb_ref[...]
    yb16 = y.astype(jnp.bfloat16)
    t = (jnp.dot(yb16, w1t_ref[...], preferred_element_type=jnp.float32)
         + b1_ref[...])
    t = t * jax.nn.sigmoid(t)
    se = jax.nn.sigmoid(
        jnp.dot(t.astype(jnp.bfloat16), w2t_ref[...],
                preferred_element_type=jnp.float32) + b2_ref[...])
    ob_ref[...] = jnp.dot((y * se).astype(jnp.bfloat16), woutt_ref[...],
                          preferred_element_type=jnp.float32)


def _stage_c(y, W1, b1, W2, b2, Wout):
    c4 = _C // 4
    return pl.pallas_call(
        _mlp_body,
        grid=(_NBLK,),
        in_specs=[
            pl.BlockSpec((_BLK, _C), lambda i: (i, 0)),
            pl.BlockSpec((_C, c4), lambda i: (0, 0)),
            pl.BlockSpec((1, c4), lambda i: (0, 0)),
            pl.BlockSpec((c4, _C), lambda i: (0, 0)),
            pl.BlockSpec((1, _C), lambda i: (0, 0)),
            pl.BlockSpec((_C, _C), lambda i: (0, 0)),
        ],
        out_specs=pl.BlockSpec((_BLK, _C), lambda i: (i, 0)),
        out_shape=jax.ShapeDtypeStruct((_L, _C), jnp.float32),
    )(y, W1.T.astype(jnp.bfloat16), b1[None, :],
      W2.T.astype(jnp.bfloat16), b2[None, :],
      Wout.T.astype(jnp.bfloat16))


def kernel(x, Ww, bw, Wq, bq, Wk, Wout, W1, b1, W2, b2):
    x2 = x.reshape(_L, _C)
    wall = jnp.zeros((_C, 256), jnp.float32)
    wall = wall.at[:, 0:3 * _H].set(Ww.T)
    wall = wall.at[:, 128:256].set(Wq.T)
    ball = jnp.zeros((256,), jnp.float32)
    ball = ball.at[0:3 * _H].set(bw)
    ball = ball.at[128:256].set(bq)
    rows = jnp.arange(_H * _POS)
    wkh = jnp.zeros((_H * _POS, _H), jnp.float32)
    wkh = wkh.at[rows, rows // _POS].set(jnp.tile(Wk[:, 0], _H)
                                         * (_POS ** -0.5))

    x_bf = x2.astype(jnp.bfloat16)
    attn8, idx, ent = _stage_a(x_bf, wall.astype(jnp.bfloat16), ball[None, :],
                               wkh.astype(jnp.bfloat16))
    x_pk = jax.lax.bitcast_convert_type(
        x_bf.reshape(_L, _C // 2, 2), jnp.int32)
    y = _stage_b(x_pk, idx, attn8)
    g = jnp.arange(_C)
    k = g % 32
    orig = (g // 32) * 32 + 2 * (k % 16) + (k // 16)
    out = _stage_c(y, W1[:, orig], b1, W2[orig, :], b2[orig], Wout[:, orig])
    return (out.reshape(_B, _L, _C), ent.reshape(()))


# speedup vs baseline: 1.0025x; 1.0025x over previous
import functools

import jax
import jax.numpy as jnp
from jax import lax
from jax.experimental import pallas as pl
from jax.experimental.pallas import tpu as pltpu
from jax.experimental.pallas import tpu_sc as plsc

_B, _L, _C = 1, 2048, 1024
_H, _POS = 8, 16
_MAXF, _MINF = 16.0, 1.0
_S = 33
_SP = 48
_IP = 40
_D = _C // _H
_BLK = 256
_NBLK = _L // _BLK
_NC, _NS = 2, 16
_NW = _NC * _NS
_RPW = _L // _NW
_HIGH = jax.lax.Precision.HIGHEST


def _attn_body(xb_ref, wall_ref, ball_ref, wkh_ref, attn_ref, idx_ref, ent_ref):
    i = pl.program_id(0)
    xb = xb_ref[...]
    p = (jnp.dot(xb, wall_ref[...], preferred_element_type=jnp.float32)
         + ball_ref[...])
    p = p * jax.nn.sigmoid(p)

    freq = jax.nn.sigmoid(p[:, 0:8]) * (_MAXF - _MINF) + _MINF
    phase = jnp.tanh(p[:, 8:16]) * _MAXF
    decay = jax.nn.sigmoid(p[:, 16:24]) * 9.5 + 0.5
    freq_avg = jnp.mean(freq, axis=1, keepdims=True)
    phase_avg = jnp.mean(phase, axis=1, keepdims=True)

    q = p[:, 128:256]
    qk = jnp.dot(q.astype(jnp.bfloat16), wkh_ref[...],
                 preferred_element_type=jnp.float32)

    centers = (jnp.float32(i * _BLK)
               + lax.broadcasted_iota(jnp.int32, (_BLK, 1), 0)
               .astype(jnp.float32))
    lane = lax.broadcasted_iota(jnp.int32, (1, _SP), 1)
    grid = lane.astype(jnp.float32) - 16.0
    sp = centers + grid * freq_avg + phase_avg
    valid = (sp >= 0.0) & (sp < float(_L)) & (lane < _S)
    idx_ref[...] = jnp.clip(sp.astype(jnp.int32), 0, _L - 1)[:, 0:_IP]

    absgrid = jnp.abs(grid)
    validf = valid.astype(jnp.float32)
    ent = jnp.zeros((1, 1), jnp.float32)
    for h in range(_H):
        rel = absgrid * freq[:, h:h + 1]
        logit = qk[:, h:h + 1] * rel
        lm = jnp.where(valid, logit, -jnp.inf)
        m = jnp.max(lm, axis=1, keepdims=True)
        e = jnp.exp(lm - m)
        a0 = e / jnp.sum(e, axis=1, keepdims=True)
        env = jnp.exp(-rel / jnp.maximum(decay[:, h:h + 1], 0.1))
        a1 = a0 * env * validf
        a = a1 / (jnp.sum(a1, axis=1, keepdims=True) + 1e-8)
        attn_ref[h] = a
        ent = ent + jnp.sum(a * jnp.log(a + 1e-8)).reshape(1, 1)

    prev = ent_ref[...]
    acc = jnp.where(i == 0, ent, prev + ent)
    ent_ref[...] = jnp.where(i == _NBLK - 1, acc * (1.0 / (_L * _H)), acc)


def _stage_a(x2, wall, ball, wkh):
    return pl.pallas_call(
        _attn_body,
        grid=(_NBLK,),
        in_specs=[
            pl.BlockSpec((_BLK, _C), lambda i: (i, 0)),
            pl.BlockSpec((_C, 256), lambda i: (0, 0)),
            pl.BlockSpec((1, 256), lambda i: (0, 0)),
            pl.BlockSpec((_D, _H), lambda i: (0, 0)),
        ],
        out_specs=[
            pl.BlockSpec((_H, _BLK, _SP), lambda ii: (0, ii, 0)),
            pl.BlockSpec((_BLK, _IP), lambda i: (i, 0)),
            pl.BlockSpec((1, 1), lambda i: (0, 0)),
        ],
        out_shape=[
            jax.ShapeDtypeStruct((_H, _L, _SP), jnp.float32),
            jax.ShapeDtypeStruct((_L, _IP), jnp.int32),
            jax.ShapeDtypeStruct((1, 1), jnp.float32),
        ],
    )(x2, wall, ball, wkh)


def _sc_combine_body(x_hbm, idx_hbm, attn_hbm, y_hbm,
                     idx_v, w_v, rows_v, out_v, sem0, sem1):
    wid = lax.axis_index("s") * _NC + lax.axis_index("c")
    base = wid * _RPW
    pltpu.sync_copy(idx_hbm.at[pl.ds(base * _IP, _RPW * _IP)], idx_v)
    for h in range(_H):
        pltpu.sync_copy(
            attn_hbm.at[pl.ds((h * _L + base) * _SP, _RPW * _SP)],
            w_v.at[pl.ds(h * _RPW * _SP, _RPW * _SP)])

    def idx_row(i):
        return idx_v.at[pl.ds(i * _IP, _IP)]

    def compute_row(i, slot):
        r = jnp.bitwise_and(i, 7)
        for h in range(_H):
            wbase = (h * _RPW) * _SP + i * _SP
            wv = [w_v[pl.ds(wbase + k * 16, 16)] for k in range(_SP // 16)]
            wsc = [wv[s // 16][s % 16] for s in range(_S)]

            def chunk(v, carry):
                offw = h * (_D // 2) + v * 16
                offo = h * _D + v * 32
                acca = [None] * 4
                accb = [None] * 4
                for s in range(_S):
                    pk = rows_v[slot, s, pl.ds(offw, 16)]
                    a = lax.bitcast_convert_type(pk << 16, jnp.float32)
                    b = lax.bitcast_convert_type(pk, jnp.float32)
                    if s < 4:
                        acca[s] = wsc[s] * a
                        accb[s] = wsc[s] * b
                    else:
                        acca[s % 4] = acca[s % 4] + wsc[s] * a
                        accb[s % 4] = accb[s % 4] + wsc[s] * b
                out_v[r, pl.ds(offo, 16)] = ((acca[0] + acca[1])
                                             + (acca[2] + acca[3]))
                out_v[r, pl.ds(offo + 16, 16)] = ((accb[0] + accb[1])
                                                  + (accb[2] + accb[3]))
                return carry

            lax.fori_loop(0, _D // 32, chunk, 0, unroll=2)

    pltpu.async_copy(x_hbm.at[idx_row(0)], rows_v.at[0], sem0)

    def body2(j, carry):
        r0 = 2 * j
        pltpu.async_copy(x_hbm.at[idx_row(r0 + 1)], rows_v.at[1], sem1)
        pltpu.make_async_copy(x_hbm.at[idx_row(r0)], rows_v.at[0],
                              sem0).wait()
        compute_row(r0, 0)

        @pl.when(j < _RPW // 2 - 1)
        def _():
            pltpu.async_copy(x_hbm.at[idx_row(r0 + 2)], rows_v.at[0], sem0)

        pltpu.make_async_copy(x_hbm.at[idx_row(r0 + 1)], rows_v.at[1],
                              sem1).wait()
        compute_row(r0 + 1, 1)

        @pl.when(jnp.bitwise_and(j, 3) == 3)
        def _():
            pltpu.sync_copy(
                out_v,
                y_hbm.at[pl.ds(pl.multiple_of(base + r0 - 6, 8), 8)])

        return carry

    lax.fori_loop(0, _RPW // 2, body2, 0)


def _stage_b(x_pk, idx, attn8):
    mesh = plsc.VectorSubcoreMesh(core_axis_name="c", subcore_axis_name="s")
    fn = functools.partial(
        pl.kernel,
        mesh=mesh,
        out_type=jax.ShapeDtypeStruct((_L, _C), jnp.float32),
        scratch_types=[
            pltpu.VMEM((_RPW * _IP,), jnp.int32),
            pltpu.VMEM((_H * _RPW * _SP,), jnp.float32),
            pltpu.VMEM((2, _IP, _C // 2), jnp.int32),
            pltpu.VMEM((8, _C), jnp.float32),
            pltpu.SemaphoreType.DMA,
            pltpu.SemaphoreType.DMA,
        ],
    )(_sc_combine_body)
    return fn(x_pk, idx.reshape(_L * _IP), attn8.reshape(_H * _L * _SP))


def _mlp_body(yb_ref, w1t_ref, b1_ref, w2t_ref, b2_ref, woutt_ref, ob_ref):
    y = yb_ref[...]
    yb16 = y.astype(jnp.bfloat16)
    t = (jnp.dot(yb16, w1t_ref[...], preferred_element_type=jnp.float32)
         + b1_ref[...])
    t = t * jax.nn.sigmoid(t)
    se = jax.nn.sigmoid(
        jnp.dot(t.astype(jnp.bfloat16), w2t_ref[...],
                preferred_element_type=jnp.float32) + b2_ref[...])
    ob_ref[...] = jnp.dot((y * se).astype(jnp.bfloat16), woutt_ref[...],
                          preferred_element_type=jnp.float32)


def _stage_c(y, W1, b1, W2, b2, Wout):
    c4 = _C // 4
    return pl.pallas_call(
        _mlp_body,
        grid=(_NBLK,),
        in_specs=[
            pl.BlockSpec((_BLK, _C), lambda i: (i, 0)),
            pl.BlockSpec((_C, c4), lambda i: (0, 0)),
            pl.BlockSpec((1, c4), lambda i: (0, 0)),
            pl.BlockSpec((c4, _C), lambda i: (0, 0)),
            pl.BlockSpec((1, _C), lambda i: (0, 0)),
            pl.BlockSpec((_C, _C), lambda i: (0, 0)),
        ],
        out_specs=pl.BlockSpec((_BLK, _C), lambda i: (i, 0)),
        out_shape=jax.ShapeDtypeStruct((_L, _C), jnp.float32),
    )(y, W1.T.astype(jnp.bfloat16), b1[None, :],
      W2.T.astype(jnp.bfloat16), b2[None, :],
      Wout.T.astype(jnp.bfloat16))


def kernel(x, Ww, bw, Wq, bq, Wk, Wout, W1, b1, W2, b2):
    x2 = x.reshape(_L, _C)
    wall = jnp.zeros((_C, 256), jnp.float32)
    wall = wall.at[:, 0:3 * _H].set(Ww.T)
    wall = wall.at[:, 128:256].set(Wq.T)
    ball = jnp.zeros((256,), jnp.float32)
    ball = ball.at[0:3 * _H].set(bw)
    ball = ball.at[128:256].set(bq)
    rows = jnp.arange(_H * _POS)
    wkh = jnp.zeros((_H * _POS, _H), jnp.float32)
    wkh = wkh.at[rows, rows // _POS].set(jnp.tile(Wk[:, 0], _H)
                                         * (_POS ** -0.5))

    x_bf = x2.astype(jnp.bfloat16)
    attn8, idx, ent = _stage_a(x_bf, wall.astype(jnp.bfloat16), ball[None, :],
                               wkh.astype(jnp.bfloat16))
    x_pk = jax.lax.bitcast_convert_type(
        x_bf.reshape(_L, _C // 2, 2), jnp.int32)
    y = _stage_b(x_pk, idx, attn8)
    g = jnp.arange(_C)
    k = g % 32
    orig = (g // 32) * 32 + 2 * (k % 16) + (k // 16)
    out = _stage_c(y, W1[:, orig], b1, W2[orig, :], b2[orig], Wout[:, orig])
    return (out.reshape(_B, _L, _C), ent.reshape(()))


# speedup vs baseline: 1.2323x; 1.2293x over previous
import functools

import jax
import jax.numpy as jnp
from jax import lax
from jax.experimental import pallas as pl
from jax.experimental.pallas import tpu as pltpu
from jax.experimental.pallas import tpu_sc as plsc

_B, _L, _C = 1, 2048, 1024
_H, _POS = 8, 16
_MAXF, _MINF = 16.0, 1.0
_S = 33
_SP = 48
_IP = 40
_D = _C // _H
_BLK = 256
_NBLK = _L // _BLK
_NC, _NS = 2, 16
_NW = _NC * _NS
_RPW = _L // _NW
_HIGH = jax.lax.Precision.HIGHEST


def _attn_body(xb_ref, wall_ref, ball_ref, wkh_ref, attn_ref, idx_ref, ent_ref):
    i = pl.program_id(0)
    xb = xb_ref[...].astype(jnp.bfloat16)
    p = (jnp.dot(xb, wall_ref[...], preferred_element_type=jnp.float32)
         + ball_ref[...])
    p = p * jax.nn.sigmoid(p)

    freq = jax.nn.sigmoid(p[:, 0:8]) * (_MAXF - _MINF) + _MINF
    phase = jnp.tanh(p[:, 8:16]) * _MAXF
    decay = jax.nn.sigmoid(p[:, 16:24]) * 9.5 + 0.5
    freq_avg = jnp.mean(freq, axis=1, keepdims=True)
    phase_avg = jnp.mean(phase, axis=1, keepdims=True)

    q = p[:, 128:256]
    qk = jnp.dot(q.astype(jnp.bfloat16), wkh_ref[...],
                 preferred_element_type=jnp.float32)

    centers = (jnp.float32(i * _BLK)
               + lax.broadcasted_iota(jnp.int32, (_BLK, 1), 0)
               .astype(jnp.float32))
    lane = lax.broadcasted_iota(jnp.int32, (1, _SP), 1)
    grid = lane.astype(jnp.float32) - 16.0
    sp = centers + grid * freq_avg + phase_avg
    valid = (sp >= 0.0) & (sp < float(_L)) & (lane < _S)
    idx_ref[...] = jnp.clip(sp.astype(jnp.int32), 0, _L - 1)[:, 0:_IP]

    absgrid = jnp.abs(grid)
    validf = valid.astype(jnp.float32)
    ent = jnp.zeros((1, 1), jnp.float32)
    for h in range(_H):
        rel = absgrid * freq[:, h:h + 1]
        logit = qk[:, h:h + 1] * rel
        lm = jnp.where(valid, logit, -jnp.inf)
        m = jnp.max(lm, axis=1, keepdims=True)
        e = jnp.exp(lm - m)
        a0 = e / jnp.sum(e, axis=1, keepdims=True)
        env = jnp.exp(-rel / jnp.maximum(decay[:, h:h + 1], 0.1))
        a1 = a0 * env * validf
        a = a1 / (jnp.sum(a1, axis=1, keepdims=True) + 1e-8)
        attn_ref[h] = a
        ent = ent + jnp.sum(a * jnp.log(a + 1e-8)).reshape(1, 1)

    prev = ent_ref[...]
    acc = jnp.where(i == 0, ent, prev + ent)
    ent_ref[...] = jnp.where(i == _NBLK - 1, acc * (1.0 / (_L * _H)), acc)


def _stage_a(x2, wall, ball, wkh):
    return pl.pallas_call(
        _attn_body,
        grid=(_NBLK,),
        in_specs=[
            pl.BlockSpec((_BLK, _C), lambda i: (i, 0)),
            pl.BlockSpec((_C, 256), lambda i: (0, 0)),
            pl.BlockSpec((1, 256), lambda i: (0, 0)),
            pl.BlockSpec((_D, _H), lambda i: (0, 0)),
        ],
        out_specs=[
            pl.BlockSpec((_H, _BLK, _SP), lambda ii: (0, ii, 0)),
            pl.BlockSpec((_BLK, _IP), lambda i: (i, 0)),
            pl.BlockSpec((1, 1), lambda i: (0, 0)),
        ],
        out_shape=[
            jax.ShapeDtypeStruct((_H, _L, _SP), jnp.float32),
            jax.ShapeDtypeStruct((_L, _IP), jnp.int32),
            jax.ShapeDtypeStruct((1, 1), jnp.float32),
        ],
    )(x2, wall, ball, wkh)


def _sc_combine_body(x_hbm, idx_hbm, attn_hbm, y_hbm,
                     idx_v, w_v, rows_v, out_v, sem0, sem1):
    wid = lax.axis_index("s") * _NC + lax.axis_index("c")
    base = wid * _RPW
    pltpu.sync_copy(idx_hbm.at[pl.ds(base * _IP, _RPW * _IP)], idx_v)
    for h in range(_H):
        pltpu.sync_copy(
            attn_hbm.at[pl.ds((h * _L + base) * _SP, _RPW * _SP)],
            w_v.at[pl.ds(h * _RPW * _SP, _RPW * _SP)])

    def idx_row(i):
        return idx_v.at[pl.ds(i * _IP, _S)]

    def compute_row(i, slot):
        r = jnp.bitwise_and(i, 7)
        for h in range(_H):
            wbase = (h * _RPW) * _SP + i * _SP
            wv = [w_v[pl.ds(wbase + k * 16, 16)] for k in range(_SP // 16)]
            wsc = [wv[s // 16][s % 16] for s in range(_S)]

            def chunk(v, carry):
                off = h * _D + v * 16
                accs = [wsc[s] * rows_v[slot, s, pl.ds(off, 16)]
                        for s in range(8)]
                for s in range(8, _S):
                    vec = rows_v[slot, s, pl.ds(off, 16)]
                    accs[s % 8] = accs[s % 8] + wsc[s] * vec
                t0 = (accs[0] + accs[1]) + (accs[2] + accs[3])
                t1 = (accs[4] + accs[5]) + (accs[6] + accs[7])
                out_v[r, pl.ds(off, 16)] = t0 + t1
                return carry

            lax.fori_loop(0, _D // 16, chunk, 0, unroll=2)

    pltpu.async_copy(x_hbm.at[idx_row(0)], rows_v.at[0], sem0)

    def body2(j, carry):
        r0 = 2 * j
        pltpu.async_copy(x_hbm.at[idx_row(r0 + 1)], rows_v.at[1], sem1)
        pltpu.make_async_copy(x_hbm.at[idx_row(r0)], rows_v.at[0],
                              sem0).wait()
        compute_row(r0, 0)

        @pl.when(j < _RPW // 2 - 1)
        def _():
            pltpu.async_copy(x_hbm.at[idx_row(r0 + 2)], rows_v.at[0], sem0)

        pltpu.make_async_copy(x_hbm.at[idx_row(r0 + 1)], rows_v.at[1],
                              sem1).wait()
        compute_row(r0 + 1, 1)

        @pl.when(jnp.bitwise_and(j, 3) == 3)
        def _():
            pltpu.sync_copy(
                out_v,
                y_hbm.at[pl.ds(pl.multiple_of(base + r0 - 6, 8), 8)])

        return carry

    lax.fori_loop(0, _RPW // 2, body2, 0)


def _stage_b(x2, idx, attn8):
    mesh = plsc.VectorSubcoreMesh(core_axis_name="c", subcore_axis_name="s")
    fn = functools.partial(
        pl.kernel,
        mesh=mesh,
        out_type=jax.ShapeDtypeStruct((_L, _C), jnp.float32),
        scratch_types=[
            pltpu.VMEM((_RPW * _IP,), jnp.int32),
            pltpu.VMEM((_H * _RPW * _SP,), jnp.float32),
            pltpu.VMEM((2, _S, _C), jnp.float32),
            pltpu.VMEM((8, _C), jnp.float32),
            pltpu.SemaphoreType.DMA,
            pltpu.SemaphoreType.DMA,
        ],
    )(_sc_combine_body)
    return fn(x2, idx.reshape(_L * _IP), attn8.reshape(_H * _L * _SP))


def _mlp_body(yb_ref, w1t_ref, b1_ref, w2t_ref, b2_ref, woutt_ref, ob_ref):
    y = yb_ref[...]
    yb16 = y.astype(jnp.bfloat16)
    t = (jnp.dot(yb16, w1t_ref[...], preferred_element_type=jnp.float32)
         + b1_ref[...])
    t = t * jax.nn.sigmoid(t)
    se = jax.nn.sigmoid(
        jnp.dot(t.astype(jnp.bfloat16), w2t_ref[...],
                preferred_element_type=jnp.float32) + b2_ref[...])
    ob_ref[...] = jnp.dot((y * se).astype(jnp.bfloat16), woutt_ref[...],
                          preferred_element_type=jnp.float32)


def _stage_c(y, W1, b1, W2, b2, Wout):
    c4 = _C // 4
    return pl.pallas_call(
        _mlp_body,
        grid=(_NBLK,),
        in_specs=[
            pl.BlockSpec((_BLK, _C), lambda i: (i, 0)),
            pl.BlockSpec((_C, c4), lambda i: (0, 0)),
            pl.BlockSpec((1, c4), lambda i: (0, 0)),
            pl.BlockSpec((c4, _C), lambda i: (0, 0)),
            pl.BlockSpec((1, _C), lambda i: (0, 0)),
            pl.BlockSpec((_C, _C), lambda i: (0, 0)),
        ],
        out_specs=pl.BlockSpec((_BLK, _C), lambda i: (i, 0)),
        out_shape=jax.ShapeDtypeStruct((_L, _C), jnp.float32),
    )(y, W1.T.astype(jnp.bfloat16), b1[None, :],
      W2.T.astype(jnp.bfloat16), b2[None, :],
      Wout.T.astype(jnp.bfloat16))


def kernel(x, Ww, bw, Wq, bq, Wk, Wout, W1, b1, W2, b2):
    x2 = x.reshape(_L, _C)
    wall = jnp.zeros((_C, 256), jnp.float32)
    wall = wall.at[:, 0:3 * _H].set(Ww.T)
    wall = wall.at[:, 128:256].set(Wq.T)
    ball = jnp.zeros((256,), jnp.float32)
    ball = ball.at[0:3 * _H].set(bw)
    ball = ball.at[128:256].set(bq)
    rows = jnp.arange(_H * _POS)
    wkh = jnp.zeros((_H * _POS, _H), jnp.float32)
    wkh = wkh.at[rows, rows // _POS].set(jnp.tile(Wk[:, 0], _H)
                                         * (_POS ** -0.5))

    attn8, idx, ent = _stage_a(x2, wall.astype(jnp.bfloat16), ball[None, :],
                               wkh.astype(jnp.bfloat16))
    y = _stage_b(x2, idx, attn8)
    out = _stage_c(y, W1, b1, W2, b2, Wout)
    return (out.reshape(_B, _L, _C), ent.reshape(()))
